# baseline (speedup 1.0000x reference)
import functools

import jax
import jax.numpy as jnp
from jax.experimental import pallas as pl
from jax.experimental.pallas import tpu as pltpu


def _body(x_ref, a_ref, ck_ref, cb_ref, dk_ref, db_ref, lk_ref, lb_ref,
          out_ref, h1_ref, m32_ref, cnt_ref, *, f_in, n_b):
    b = pl.program_id(0)
    hdim = h1_ref.shape[2]

    @pl.when(b == 0)
    def _prologue():
        ones_row = jnp.ones((1, hdim), dtype=jnp.float32)

        def _pre(g, _):
            h1_ref[g] = jnp.dot(x_ref[g, :, :f_in], ck_ref[...],
                                preferred_element_type=jnp.float32)
            mrep = jnp.dot(x_ref[g, :, f_in:f_in + 1], ones_row,
                           preferred_element_type=jnp.float32)
            m32 = (mrep != 0.0).astype(jnp.float32)
            m32_ref[g] = m32
            cnt_ref[g, 0] = jnp.sum(m32) / hdim
            return 0
        jax.lax.fori_loop(0, n_b, _pre, 0)

    z = jnp.dot(a_ref[0], h1_ref[b], preferred_element_type=jnp.float32)

    m32 = m32_ref[b]
    ssum = jnp.sum(z * m32, axis=0, keepdims=True)
    smax = jnp.max(jnp.where(m32 != 0.0, z, -jnp.inf),
                   axis=0, keepdims=True)
    cnt = cnt_ref[b, 0]

    avg = ssum / jnp.maximum(cnt, 1.0) + cb_ref[...] * jnp.minimum(cnt, 1.0)
    smax = jnp.maximum(smax + cb_ref[...], -1e9)
    pooled = jnp.concatenate([avg, smax], axis=1)
    hid = jnp.dot(pooled, dk_ref[...],
                  preferred_element_type=jnp.float32) + db_ref[...]
    hid = jnp.maximum(hid, 0.0)
    out = jnp.dot(hid, lk_ref[...],
                  preferred_element_type=jnp.float32) + lb_ref[...]
    out_ref[0] = out


@jax.jit
def kernel(x, a, conv1_kernel, conv1_bias, dense1_kernel, dense1_bias,
           last_kernel, last_bias):
    B, N, fp1 = x.shape
    f_in = fp1 - 1
    hdim = conv1_kernel.shape[1]
    n_hidden = dense1_kernel.shape[1]
    n_labels = last_kernel.shape[1]

    cb = conv1_bias.reshape(1, hdim)
    db = dense1_bias.reshape(1, n_hidden)
    lb = last_bias.reshape(1, n_labels)

    out = pl.pallas_call(
        functools.partial(_body, f_in=f_in, n_b=B),
        grid=(B,),
        in_specs=[
            pl.BlockSpec((B, N, fp1), lambda b: (0, 0, 0)),
            pl.BlockSpec((1, N, N), lambda b: (b, 0, 0)),
            pl.BlockSpec((f_in, hdim), lambda b: (0, 0)),
            pl.BlockSpec((1, hdim), lambda b: (0, 0)),
            pl.BlockSpec((2 * hdim, n_hidden), lambda b: (0, 0)),
            pl.BlockSpec((1, n_hidden), lambda b: (0, 0)),
            pl.BlockSpec((n_hidden, n_labels), lambda b: (0, 0)),
            pl.BlockSpec((1, n_labels), lambda b: (0, 0)),
        ],
        out_specs=pl.BlockSpec((1, 1, n_labels), lambda b: (b, 0, 0)),
        out_shape=jax.ShapeDtypeStruct((B, 1, n_labels), jnp.float32),
        scratch_shapes=[
            pltpu.VMEM((B, N, hdim), jnp.float32),
            pltpu.VMEM((B, N, hdim), jnp.float32),
            pltpu.SMEM((B, 1), jnp.float32),
        ],
        compiler_params=pltpu.CompilerParams(
            dimension_semantics=("arbitrary",),
        ),
    )(x, a, conv1_kernel, cb, dense1_kernel, db, last_kernel, lb)
    return out.reshape(B, n_labels)


# speedup vs baseline: 1.0571x; 1.0571x over previous
import functools

import jax
import jax.numpy as jnp
from jax.experimental import pallas as pl
from jax.experimental.pallas import tpu as pltpu


def _body(x_ref, a_ref, ck_ref, cb_ref, dk_ref, db_ref, lk_ref, lb_ref,
          out_ref, *, f_in):
    b = pl.program_id(0)
    h1 = jnp.dot(x_ref[b, :, :f_in], ck_ref[...],
                 preferred_element_type=jnp.float32)
    z = jnp.dot(a_ref[0], h1, preferred_element_type=jnp.float32)

    mcol = x_ref[b, :, f_in:f_in + 1]
    valid = mcol != 0.0
    m01 = valid.astype(jnp.float32)
    cnt = jnp.sum(m01)
    ssum = jnp.sum(z * m01, axis=0, keepdims=True)
    smax = jnp.max(jnp.where(valid, z, -jnp.inf), axis=0,
                   keepdims=True)

    avg = ssum / jnp.maximum(cnt, 1.0) + cb_ref[...] * jnp.minimum(cnt, 1.0)
    smax = jnp.maximum(smax + cb_ref[...], -1e9)
    pooled = jnp.concatenate([avg, smax], axis=1)
    hid = jnp.dot(pooled, dk_ref[...],
                  preferred_element_type=jnp.float32) + db_ref[...]
    hid = jnp.maximum(hid, 0.0)
    out = jnp.dot(hid, lk_ref[...],
                  preferred_element_type=jnp.float32) + lb_ref[...]
    out_ref[0] = out


@jax.jit
def kernel(x, a, conv1_kernel, conv1_bias, dense1_kernel, dense1_bias,
           last_kernel, last_bias):
    B, N, fp1 = x.shape
    f_in = fp1 - 1
    hdim = conv1_kernel.shape[1]
    n_hidden = dense1_kernel.shape[1]
    n_labels = last_kernel.shape[1]

    cb = conv1_bias.reshape(1, hdim)
    db = dense1_bias.reshape(1, n_hidden)
    lb = last_bias.reshape(1, n_labels)

    out = pl.pallas_call(
        functools.partial(_body, f_in=f_in),
        grid=(B,),
        in_specs=[
            pl.BlockSpec((B, N, fp1), lambda b: (0, 0, 0)),
            pl.BlockSpec((1, N, N), lambda b: (b, 0, 0)),
            pl.BlockSpec((f_in, hdim), lambda b: (0, 0)),
            pl.BlockSpec((1, hdim), lambda b: (0, 0)),
            pl.BlockSpec((2 * hdim, n_hidden), lambda b: (0, 0)),
            pl.BlockSpec((1, n_hidden), lambda b: (0, 0)),
            pl.BlockSpec((n_hidden, n_labels), lambda b: (0, 0)),
            pl.BlockSpec((1, n_labels), lambda b: (0, 0)),
        ],
        out_specs=pl.BlockSpec((1, 1, n_labels), lambda b: (b, 0, 0)),
        out_shape=jax.ShapeDtypeStruct((B, 1, n_labels), jnp.float32),
        compiler_params=pltpu.CompilerParams(
            dimension_semantics=("parallel",),
        ),
    )(x, a, conv1_kernel, cb, dense1_kernel, db, last_kernel, lb)
    return out.reshape(B, n_labels)


# speedup vs baseline: 1.0638x; 1.0063x over previous
import functools

import jax
import jax.numpy as jnp
from jax.experimental import pallas as pl
from jax.experimental.pallas import tpu as pltpu


def _body(x_ref, a_ref, ck_ref, cb_ref, dk_ref, db_ref, lk_ref, lb_ref,
          out_ref, *, f_in):
    h1 = jnp.dot(x_ref[0, :, :f_in], ck_ref[...],
                 preferred_element_type=jnp.float32)
    z = jnp.dot(a_ref[0], h1, preferred_element_type=jnp.float32)

    mcol = x_ref[0, :, f_in:f_in + 1]
    valid = mcol != 0.0
    m01 = valid.astype(jnp.float32)
    cnt = jnp.sum(m01)
    ssum = jnp.sum(z * m01, axis=0, keepdims=True)
    smax = jnp.max(jnp.where(valid, z, -jnp.inf), axis=0,
                   keepdims=True)

    avg = ssum / jnp.maximum(cnt, 1.0) + cb_ref[...] * jnp.minimum(cnt, 1.0)
    smax = jnp.maximum(smax + cb_ref[...], -1e9)
    pooled = jnp.concatenate([avg, smax], axis=1)
    hid = jnp.dot(pooled, dk_ref[...],
                  preferred_element_type=jnp.float32) + db_ref[...]
    hid = jnp.maximum(hid, 0.0)
    out = jnp.dot(hid, lk_ref[...],
                  preferred_element_type=jnp.float32) + lb_ref[...]
    out_ref[0] = out


@jax.jit
def kernel(x, a, conv1_kernel, conv1_bias, dense1_kernel, dense1_bias,
           last_kernel, last_bias):
    B, N, fp1 = x.shape
    f_in = fp1 - 1
    hdim = conv1_kernel.shape[1]
    n_hidden = dense1_kernel.shape[1]
    n_labels = last_kernel.shape[1]

    cb = conv1_bias.reshape(1, hdim)
    db = dense1_bias.reshape(1, n_hidden)
    lb = last_bias.reshape(1, n_labels)

    out = pl.pallas_call(
        functools.partial(_body, f_in=f_in),
        grid=(B,),
        in_specs=[
            pl.BlockSpec((1, N, fp1), lambda b: (b, 0, 0)),
            pl.BlockSpec((1, N, N), lambda b: (b, 0, 0)),
            pl.BlockSpec((f_in, hdim), lambda b: (0, 0)),
            pl.BlockSpec((1, hdim), lambda b: (0, 0)),
            pl.BlockSpec((2 * hdim, n_hidden), lambda b: (0, 0)),
            pl.BlockSpec((1, n_hidden), lambda b: (0, 0)),
            pl.BlockSpec((n_hidden, n_labels), lambda b: (0, 0)),
            pl.BlockSpec((1, n_labels), lambda b: (0, 0)),
        ],
        out_specs=pl.BlockSpec((1, 1, n_labels), lambda b: (b, 0, 0)),
        out_shape=jax.ShapeDtypeStruct((B, 1, n_labels), jnp.float32),
        compiler_params=pltpu.CompilerParams(
            dimension_semantics=("arbitrary",),
        ),
    )(x, a, conv1_kernel, cb, dense1_kernel, db, last_kernel, lb)
    return out.reshape(B, n_labels)
